# baseline (speedup 1.0000x reference)
import functools

import jax
import jax.numpy as jnp
from jax import lax
from jax.experimental import pallas as pl
from jax.experimental.pallas import tpu as pltpu
from jax.experimental.pallas import tpu_sc as plsc

V, D, B, L = 1000000, 64, 16384, 50

NC, NS = 2, 16
NW = NC * NS
NB = B // NW
PIPE = 4
GROUPS = NB // 16


def _sc_pool_body(x_hbm, emb_hbm, w_hbm, fcb_hbm, out_hbm,
                  x_v, buf0, buf1, buf2, buf3, out_v, w_v, fcb_v,
                  sem0, sem1, sem2, sem3):
    wid = lax.axis_index("s") * NC + lax.axis_index("c")
    base = wid * NB

    pltpu.sync_copy(x_hbm.at[pl.ds(base, NB), :], x_v)
    pltpu.sync_copy(w_hbm, w_v)
    pltpu.sync_copy(fcb_hbm, fcb_v)

    w4 = [w_v[0, pl.ds(16 * k, 16)] for k in range(4)]
    fcb16 = fcb_v[...]
    lane = lax.iota(jnp.int32, 16)
    lane15 = lane == 15

    bufs = [buf0, buf1, buf2, buf3]
    sems = [sem0, sem1, sem2, sem3]

    def _fire(b, buf, sem):
        pltpu.async_copy(emb_hbm.at[x_v.at[b]], buf, sem)

    def _drain(buf, sem):
        pltpu.make_async_copy(
            emb_hbm.at[x_v.at[0]], buf, sem).wait()

    def _process(b, buf):
        acc = [jnp.zeros((16,), jnp.float32) for _ in range(4)]
        for l in range(L):
            for k in range(4):
                acc[k] = acc[k] + buf[l, pl.ds(16 * k, 16)] * w4[k]
        s = (acc[0] + acc[1]) + (acc[2] + acc[3])
        cum = plsc.cumsum(s)
        plsc.store_scatter(out_v, [jnp.full((16,), b, jnp.int32)],
                           cum, mask=lane15)

    for q in range(PIPE):
        _fire(q, bufs[q], sems[q])

    def quad_body(qq, carry):
        b0 = qq * PIPE
        for u in range(PIPE):
            b = b0 + u
            _drain(bufs[u], sems[u])
            _process(b, bufs[u])

            @pl.when(b + PIPE < NB)
            def _():
                _fire(b + PIPE, bufs[u], sems[u])
        return carry

    lax.fori_loop(0, NB // PIPE, quad_body, 0)

    def fin_body(g, carry):
        rows = g * 16 + lane
        cnt = jnp.zeros((16,), jnp.float32)
        one = jnp.ones((16,), jnp.float32)
        zero = jnp.zeros((16,), jnp.float32)
        for l in range(L):
            tok = plsc.load_gather(x_v, [rows, jnp.full((16,), l, jnp.int32)])
            cnt = cnt + jnp.where(tok != 0, one, zero)
        raw = out_v[pl.ds(g * 16, 16)]
        out_v[pl.ds(g * 16, 16)] = raw / jnp.maximum(cnt, one) + fcb16
        return carry

    lax.fori_loop(0, GROUPS, fin_body, 0)
    pltpu.sync_copy(out_v, out_hbm.at[pl.ds(base, NB)])


@functools.lru_cache(maxsize=1)
def _make_sc_pool():
    mesh = plsc.VectorSubcoreMesh(
        core_axis_name="c", subcore_axis_name="s", num_cores=NC)
    return pl.kernel(
        _sc_pool_body,
        out_type=jax.ShapeDtypeStruct((B,), jnp.float32),
        mesh=mesh,
        scratch_types=[
            pltpu.VMEM((NB, L), jnp.int32),
            pltpu.VMEM((L, D), jnp.float32),
            pltpu.VMEM((L, D), jnp.float32),
            pltpu.VMEM((L, D), jnp.float32),
            pltpu.VMEM((L, D), jnp.float32),
            pltpu.VMEM((NB,), jnp.float32),
            pltpu.VMEM((1, D), jnp.float32),
            pltpu.VMEM((16,), jnp.float32),
            pltpu.SemaphoreType.DMA,
            pltpu.SemaphoreType.DMA,
            pltpu.SemaphoreType.DMA,
            pltpu.SemaphoreType.DMA,
        ],
        compiler_params=pltpu.CompilerParams(
            needs_layout_passes=False, use_tc_tiling_on_sc=False),
    )


def kernel(x, emb, fc_w, fc_b):
    fcb16 = jnp.broadcast_to(fc_b.astype(jnp.float32), (16,))
    return _make_sc_pool()(x, emb, fc_w, fcb16)
